# baseline (speedup 1.0000x reference)
import jax
import jax.numpy as jnp
from jax import lax
from jax.experimental import pallas as pl
from jax.experimental.pallas import tpu as pltpu
from jax.experimental.pallas import tpu_sc as plsc

BATCH = 16384
DIM = 64
N_CLASSES = 1000000
NUM_CORES = 2
NUM_SUBCORES = 16
NUM_WORKERS = NUM_CORES * NUM_SUBCORES
COLS_PER_WORKER = BATCH // NUM_WORKERS
LANES = 16
LANE_TILE = 128
TCOLS = (N_CLASSES + LANE_TILE - 1) // LANE_TILE
TQ = TCOLS // NUM_WORKERS
TR = TCOLS % NUM_WORKERS
CHUNK_TC = 4
CHUNK_L = CHUNK_TC * LANE_TILE
FLUSH = 128
STAGE_ROWS = BATCH + FLUSH
WL_CAP = BATCH + LANES
NGRP = BATCH // LANES


def _iota16():
    return lax.iota(jnp.int32, LANES)


def _a_body(cls_hbm, pt_hbm, stage_hbm,
            cls_all, wl_z, wl_n, chunk, outbuf, gz, gn, nlist, sem):
    w = lax.axis_index("s") * NUM_CORES + lax.axis_index("c")
    lo = w * TQ + jnp.minimum(w, TR)
    cnt = TQ + jnp.where(w < TR, 1, 0)
    nchunks = (cnt + CHUNK_TC - 1) // CHUNK_TC
    zlo = lo * LANE_TILE
    zhi = jnp.minimum((lo + cnt) * LANE_TILE, N_CLASSES)

    pltpu.sync_copy(cls_hbm, cls_all.at[pl.ds(0, BATCH)])

    def init_body(g, carry):
        wl_z[pl.ds(g * LANES, LANES)] = jnp.full((LANES,), 1 << 30, jnp.int32)
        return carry

    lax.fori_loop(0, WL_CAP // LANES + 1, init_body, 0)

    def scan_body(g, off):
        vz = cls_all[pl.ds(g * LANES, LANES)]
        vn = _iota16() + g * LANES
        mask = (vz >= zlo) & (vz < zhi)
        mi = plsc.cumsum(mask.astype(jnp.int32))
        slots = mi + (off - 1)
        plsc.store_scatter(wl_z, [slots], vz, mask=mask)
        plsc.store_scatter(wl_n, [slots], vn, mask=mask)
        return off + mi[LANES - 1]

    wl_len = lax.fori_loop(0, NGRP, scan_body, 0)
    ngrp_wl = (wl_len + LANES - 1) // LANES

    def preset_nlist():
        for k in range(FLUSH // LANES):
            nlist[pl.ds(k * LANES, LANES)] = (
                _iota16() + (BATCH + k * LANES)
            )

    preset_nlist()

    def flush():
        pltpu.async_copy(outbuf, stage_hbm.at[nlist], sem).wait()
        preset_nlist()

    def chunk_body(i, s):
        L0 = pl.multiple_of((lo + i * CHUNK_TC) * LANE_TILE, LANE_TILE)
        pltpu.sync_copy(pt_hbm.at[:, pl.ds(L0, CHUNK_L)], chunk)

        def grp_body(g, s2):
            vz = wl_z[pl.ds(g * LANES, LANES)]
            vn = wl_n[pl.ds(g * LANES, LANES)]
            mask = (vz >= L0) & (vz < L0 + CHUNK_L)
            mi = plsc.cumsum(mask.astype(jnp.int32))
            slots = mi - 1
            plsc.store_scatter(gz, [slots], vz - L0, mask=mask)
            plsc.store_scatter(gn, [slots], vn, mask=mask)
            mg = mi[LANES - 1]

            def match_body(t, s3):
                zl = gz[pl.ds(t, LANES)][0]
                n = gn[pl.ds(t, LANES)][0]
                zvec = jnp.full((LANES,), zl, jnp.int32)
                for j in range(DIM // LANES):
                    dvec = _iota16() + j * LANES
                    v = plsc.load_gather(chunk, [dvec, zvec])
                    outbuf[s3, pl.ds(j * LANES, LANES)] = v
                plsc.store_scatter(
                    nlist,
                    [jnp.full((LANES,), s3, jnp.int32)],
                    jnp.full((LANES,), n, jnp.int32),
                    mask=_iota16() == 0,
                )
                s4 = s3 + 1

                def do_flush():
                    flush()

                @pl.when(s4 == FLUSH)
                def _():
                    do_flush()

                return jnp.where(s4 == FLUSH, 0, s4)

            return lax.fori_loop(0, mg, match_body, s2)

        return lax.fori_loop(0, ngrp_wl, grp_body, s)

    s_end = lax.fori_loop(0, nchunks, chunk_body, 0)

    @pl.when(s_end > 0)
    def _():
        flush()


def _b_body(xt_hbm, stage_hbm, out_hbm, acc, piece):
    w = lax.axis_index("s") * NUM_CORES + lax.axis_index("c")
    base = pl.multiple_of(w * COLS_PER_WORKER, COLS_PER_WORKER)
    pltpu.sync_copy(xt_hbm.at[:, pl.ds(base, COLS_PER_WORKER)], acc)
    half = COLS_PER_WORKER // 2
    for p in range(2):
        pltpu.sync_copy(
            stage_hbm.at[pl.ds(base + p * half, half), :], piece
        )

        def row_body(r, carry):
            rvec = jnp.full((LANES,), r, jnp.int32)
            cvec = jnp.full((LANES,), p * half + r, jnp.int32)
            for j in range(DIM // LANES):
                dvec = _iota16() + j * LANES
                v = plsc.load_gather(piece, [rvec, dvec])
                plsc.addupdate_scatter(acc, [dvec, cvec], v)
            return carry

        lax.fori_loop(0, half, row_body, 0)
    pltpu.sync_copy(acc, out_hbm.at[:, pl.ds(base, COLS_PER_WORKER)])


@jax.jit
def kernel(x, classes, biases):
    cls32 = classes.astype(jnp.int32)
    mesh = plsc.VectorSubcoreMesh(core_axis_name="c", subcore_axis_name="s")
    run_a = pl.kernel(
        _a_body,
        out_type=jax.ShapeDtypeStruct((STAGE_ROWS, LANE_TILE), jnp.float32),
        mesh=mesh,
        scratch_types=[
            pltpu.VMEM((WL_CAP,), jnp.int32),
            pltpu.VMEM((WL_CAP,), jnp.int32),
            pltpu.VMEM((WL_CAP,), jnp.int32),
            pltpu.VMEM((DIM, CHUNK_L), jnp.float32),
            pltpu.VMEM((FLUSH, LANE_TILE), jnp.float32),
            pltpu.VMEM((2 * LANES,), jnp.int32),
            pltpu.VMEM((2 * LANES,), jnp.int32),
            pltpu.VMEM((FLUSH,), jnp.int32),
            pltpu.SemaphoreType.DMA,
        ],
        compiler_params=pltpu.CompilerParams(needs_layout_passes=False),
    )
    staging = run_a(cls32, biases.T)

    run_b = pl.kernel(
        _b_body,
        out_type=jax.ShapeDtypeStruct((DIM, BATCH), jnp.float32),
        mesh=mesh,
        scratch_types=[
            pltpu.VMEM((DIM, COLS_PER_WORKER), jnp.float32),
            pltpu.VMEM((COLS_PER_WORKER // 2, LANE_TILE), jnp.float32),
        ],
        compiler_params=pltpu.CompilerParams(needs_layout_passes=False),
    )
    out_t = run_b(x.T, staging)
    return out_t.T


# speedup vs baseline: 1.3898x; 1.3898x over previous
import jax
import jax.numpy as jnp
from jax import lax
from jax.experimental import pallas as pl
from jax.experimental.pallas import tpu as pltpu
from jax.experimental.pallas import tpu_sc as plsc

BATCH = 16384
DIM = 64
N_CLASSES = 1000000
NUM_CORES = 2
NUM_SUBCORES = 16
NUM_WORKERS = NUM_CORES * NUM_SUBCORES
COLS_PER_WORKER = BATCH // NUM_WORKERS
LANES = 16
LANE_TILE = 128
TCOLS = (N_CLASSES + LANE_TILE - 1) // LANE_TILE
TQ = TCOLS // NUM_WORKERS
TR = TCOLS % NUM_WORKERS
CHUNK_TC = 4
CHUNK_L = CHUNK_TC * LANE_TILE
FLUSH = 128
STAGE_ROWS = BATCH + FLUSH
WL_CAP = BATCH + LANES
NGRP = BATCH // LANES
SENTINEL = 1 << 30


def _iota16():
    return lax.iota(jnp.int32, LANES)


def _a_body(cls_hbm, pt_hbm, stage_hbm,
            cls_all, wl_n, chunk, outbuf, gn, nlist, sem, csem):
    w = lax.axis_index("s") * NUM_CORES + lax.axis_index("c")
    lo = w * TQ + jnp.minimum(w, TR)
    cnt = TQ + jnp.where(w < TR, 1, 0)
    nchunks = (cnt + CHUNK_TC - 1) // CHUNK_TC
    zlo = lo * LANE_TILE
    zhi = jnp.minimum((lo + cnt) * LANE_TILE, N_CLASSES)

    pltpu.sync_copy(cls_hbm, cls_all.at[pl.ds(0, BATCH)])
    cls_all[pl.ds(BATCH, LANES)] = jnp.full((LANES,), SENTINEL, jnp.int32)

    def init_body(g, carry):
        wl_n[pl.ds(g * LANES, LANES)] = jnp.full((LANES,), BATCH, jnp.int32)
        return carry

    lax.fori_loop(0, WL_CAP // LANES, init_body, 0)

    def scan_body(g, off):
        vz = cls_all[pl.ds(g * LANES, LANES)]
        vn = _iota16() + g * LANES
        mask = (vz >= zlo) & (vz < zhi)
        mi = plsc.cumsum(mask.astype(jnp.int32))
        slots = mi + (off - 1)
        plsc.store_scatter(wl_n, [slots], vn, mask=mask)
        return off + mi[LANES - 1]

    wl_len = lax.fori_loop(0, NGRP, scan_body, 0)
    ngrp_wl = (wl_len + LANES - 1) // LANES

    def preset_nlist():
        for k in range(FLUSH // LANES):
            nlist[pl.ds(k * LANES, LANES)] = _iota16() + (BATCH + k * LANES)

    preset_nlist()

    def flush():
        pltpu.async_copy(outbuf, stage_hbm.at[nlist], sem).wait()
        preset_nlist()

    def fetch_chunk(i, k):
        L0 = pl.multiple_of((lo + i * CHUNK_TC) * LANE_TILE, LANE_TILE)
        pltpu.async_copy(
            pt_hbm.at[:, pl.ds(L0, CHUNK_L)], chunk.at[k], csem.at[k]
        )

    def wait_chunk(k):
        pltpu.make_async_copy(
            pt_hbm.at[:, pl.ds(0, CHUNK_L)], chunk.at[k], csem.at[k]
        ).wait()

    fetch_chunk(0, 0)

    def chunk_body(i, s):
        k = lax.rem(i, 2)
        wait_chunk(k)

        @pl.when(i + 1 < nchunks)
        def _():
            fetch_chunk(i + 1, 1 - k)

        L0 = (lo + i * CHUNK_TC) * LANE_TILE
        kvec = jnp.full((LANES,), k, jnp.int32)

        def grp_body(g, s2):
            vn = wl_n[pl.ds(g * LANES, LANES)]
            vz = plsc.load_gather(cls_all, [vn])
            mask = (vz >= L0) & (vz < L0 + CHUNK_L)
            mi = plsc.cumsum(mask.astype(jnp.int32))
            slots = mi - 1
            plsc.store_scatter(gn, [slots], vn, mask=mask)
            mg = mi[LANES - 1]

            def match_body(t, s3):
                n = gn[pl.ds(t, LANES)][0]
                zl = cls_all[pl.ds(n, LANES)][0] - L0
                zvec = jnp.full((LANES,), zl, jnp.int32)
                for j in range(DIM // LANES):
                    dvec = _iota16() + j * LANES
                    v = plsc.load_gather(chunk, [kvec, dvec, zvec])
                    outbuf[s3, pl.ds(j * LANES, LANES)] = v
                plsc.store_scatter(
                    nlist,
                    [jnp.full((LANES,), s3, jnp.int32)],
                    jnp.full((LANES,), n, jnp.int32),
                    mask=_iota16() == 0,
                )
                s4 = s3 + 1

                @pl.when(s4 == FLUSH)
                def _():
                    flush()

                return jnp.where(s4 == FLUSH, 0, s4)

            return lax.fori_loop(0, mg, match_body, s2)

        return lax.fori_loop(0, ngrp_wl, grp_body, s)

    s_end = lax.fori_loop(0, nchunks, chunk_body, 0)

    @pl.when(s_end > 0)
    def _():
        flush()


def _b_body(xt_hbm, stage_hbm, out_hbm, acc, piece):
    w = lax.axis_index("s") * NUM_CORES + lax.axis_index("c")
    base = pl.multiple_of(w * COLS_PER_WORKER, COLS_PER_WORKER)
    pltpu.sync_copy(xt_hbm.at[:, pl.ds(base, COLS_PER_WORKER)], acc)
    half = COLS_PER_WORKER // 2
    for p in range(2):
        pltpu.sync_copy(
            stage_hbm.at[pl.ds(base + p * half, half), :], piece
        )

        def row_body(r, carry):
            rvec = jnp.full((LANES,), r, jnp.int32)
            cvec = jnp.full((LANES,), p * half + r, jnp.int32)
            for j in range(DIM // LANES):
                dvec = _iota16() + j * LANES
                v = plsc.load_gather(piece, [rvec, dvec])
                plsc.addupdate_scatter(acc, [dvec, cvec], v)
            return carry

        lax.fori_loop(0, half, row_body, 0)
    pltpu.sync_copy(acc, out_hbm.at[:, pl.ds(base, COLS_PER_WORKER)])


@jax.jit
def kernel(x, classes, biases):
    cls32 = classes.astype(jnp.int32)
    mesh = plsc.VectorSubcoreMesh(core_axis_name="c", subcore_axis_name="s")
    run_a = pl.kernel(
        _a_body,
        out_type=jax.ShapeDtypeStruct((STAGE_ROWS, LANE_TILE), jnp.float32),
        mesh=mesh,
        scratch_types=[
            pltpu.VMEM((WL_CAP,), jnp.int32),
            pltpu.VMEM((WL_CAP,), jnp.int32),
            pltpu.VMEM((2, DIM, CHUNK_L), jnp.float32),
            pltpu.VMEM((FLUSH, LANE_TILE), jnp.float32),
            pltpu.VMEM((2 * LANES,), jnp.int32),
            pltpu.VMEM((FLUSH,), jnp.int32),
            pltpu.SemaphoreType.DMA,
            pltpu.SemaphoreType.DMA((2,)),
        ],
        compiler_params=pltpu.CompilerParams(needs_layout_passes=False),
    )
    staging = run_a(cls32, biases.T)

    run_b = pl.kernel(
        _b_body,
        out_type=jax.ShapeDtypeStruct((DIM, BATCH), jnp.float32),
        mesh=mesh,
        scratch_types=[
            pltpu.VMEM((DIM, COLS_PER_WORKER), jnp.float32),
            pltpu.VMEM((COLS_PER_WORKER // 2, LANE_TILE), jnp.float32),
        ],
        compiler_params=pltpu.CompilerParams(needs_layout_passes=False),
    )
    out_t = run_b(x.T, staging)
    return out_t.T
